# baseline (speedup 1.0000x reference)
import functools

import jax
import jax.numpy as jnp
from jax import lax
from jax.experimental import pallas as pl
from jax.experimental.pallas import tpu as pltpu
from jax.experimental.pallas import tpu_sc as plsc

N = 10000
E = 320000
C = 128
D_EDGE = 4
D_OUT = 2

NC = 2
NS = 16
NW = NC * NS
KA = 128
KB = 64
EPWA0 = 12288
EPWA1 = 8192
EPWB0 = 13824
EPWB1 = 6656
EP = NS * (EPWA0 + EPWA1)
NPAD = 10240
NSL = NPAD // NS

_f32 = jnp.float32
_i32 = jnp.int32


def _tc_xlxr(x, W_l, W_r):
    def body(x_ref, wl_ref, wr_ref, xl_ref, xr_ref):
        xb = x_ref[...]
        xl_ref[...] = jnp.dot(xb, wl_ref[...], preferred_element_type=_f32)
        xr_ref[...] = jnp.dot(xb, wr_ref[...], preferred_element_type=_f32)

    return pl.pallas_call(
        body,
        grid=(10,),
        in_specs=[
            pl.BlockSpec((1000, C), lambda i: (i, 0)),
            pl.BlockSpec((C, C), lambda i: (0, 0)),
            pl.BlockSpec((C, C), lambda i: (0, 0)),
        ],
        out_specs=[
            pl.BlockSpec((1000, C), lambda i: (i, 0)),
            pl.BlockSpec((1000, C), lambda i: (i, 0)),
        ],
        out_shape=[
            jax.ShapeDtypeStruct((N, C), _f32),
            jax.ShapeDtypeStruct((N, C), _f32),
        ],
    )(x, W_l, W_r)


def _tc_ea(efp, W_e):
    def body(ef_ref, we_ref, ea_ref):
        ea_ref[...] = jnp.dot(ef_ref[...], we_ref[...],
                              preferred_element_type=_f32)

    return pl.pallas_call(
        body,
        grid=(80,),
        in_specs=[
            pl.BlockSpec((EP // 80, D_EDGE), lambda i: (i, 0)),
            pl.BlockSpec((D_EDGE, C), lambda i: (0, 0)),
        ],
        out_specs=pl.BlockSpec((EP // 80, C), lambda i: (i, 0)),
        out_shape=jax.ShapeDtypeStruct((EP, C), _f32),
    )(efp, W_e)


def _tc_mid(hp, b_gat, W_gcn, denp):
    def body(hp_ref, bg_ref, wg_ref, dp_ref, hw_ref, dis_ref):
        h = jnp.maximum(hp_ref[0] + hp_ref[1] + bg_ref[...][None, :], 0.0)
        hw_ref[...] = jnp.dot(h, wg_ref[...], preferred_element_type=_f32)
        den = dp_ref[0] + dp_ref[1]
        dis_ref[...] = jnp.where(den > 0, 1.0, 0.0)

    return pl.pallas_call(
        body,
        grid=(10,),
        in_specs=[
            pl.BlockSpec((2, 1024, C), lambda i: (0, i, 0)),
            pl.BlockSpec((C,), lambda i: (0,)),
            pl.BlockSpec((C, C), lambda i: (0, 0)),
            pl.BlockSpec((2, 1024), lambda i: (0, i)),
        ],
        out_specs=[
            pl.BlockSpec((1024, C), lambda i: (i, 0)),
            pl.BlockSpec((1024,), lambda i: (i,)),
        ],
        out_shape=[
            jax.ShapeDtypeStruct((NPAD, C), _f32),
            jax.ShapeDtypeStruct((NPAD,), _f32),
        ],
    )(hp, b_gat, W_gcn, denp)


def _tc_out(h2p, b_gcn, W_out, b_out):
    def body(h2p_ref, bg_ref, wo_ref, bo_ref, out_ref):
        h2 = jnp.maximum(h2p_ref[0] + h2p_ref[1] + bg_ref[...][None, :], 0.0)
        out_ref[...] = (jnp.dot(h2, wo_ref[...], preferred_element_type=_f32)
                        + bo_ref[...][None, :])

    return pl.pallas_call(
        body,
        grid=(10,),
        in_specs=[
            pl.BlockSpec((2, 1024, C), lambda i: (0, i, 0)),
            pl.BlockSpec((C,), lambda i: (0,)),
            pl.BlockSpec((C, D_OUT), lambda i: (0, 0)),
            pl.BlockSpec((D_OUT,), lambda i: (0,)),
        ],
        out_specs=pl.BlockSpec((1024, D_OUT), lambda i: (i, 0)),
        out_shape=jax.ShapeDtypeStruct((NPAD, D_OUT), _f32),
    )(h2p, b_gcn, W_out, b_out)


def _sc_mesh():
    return plsc.VectorSubcoreMesh(core_axis_name="c", subcore_axis_name="s",
                                  num_cores=NC, num_subcores=NS)


_SC_PARAMS = pltpu.CompilerParams(needs_layout_passes=False)


def _core_part(c, s, k, epw0, epw1):
    base = jnp.where(c == 0, s * epw0, NS * epw0 + s * epw1)
    nch = jnp.where(c == 0, epw0 // k, epw1 // k)
    return base, nch


def _sc_pass_a(src, dst, xl, xr, ea, attf, zn):
    @functools.partial(
        pl.kernel,
        mesh=_sc_mesh(),
        compiler_params=_SC_PARAMS,
        out_type=(
            jax.ShapeDtypeStruct((EP,), _f32),
            jax.ShapeDtypeStruct((NC, NPAD), _f32),
        ),
        scratch_types=[
            [pltpu.VMEM((KA,), _i32)] * 4,
            [pltpu.VMEM((KA,), _i32)] * 4,
            [pltpu.VMEM((KA, C), _f32)] * 2,
            [pltpu.VMEM((KA, C), _f32)] * 2,
            [pltpu.VMEM((KA, C), _f32)] * 2,
            [pltpu.VMEM((KA,), _f32)] * 2,
            pltpu.VMEM((KA * 16,), _f32),
            pltpu.VMEM((C,), _f32),
            pltpu.VMEM_SHARED((NPAD,), _f32),
            [pltpu.SemaphoreType.DMA] * 4,
            [pltpu.SemaphoreType.DMA] * 2,
            [pltpu.SemaphoreType.DMA] * 2,
            [pltpu.SemaphoreType.DMA] * 2,
        ],
    )
    def sca(src_hbm, dst_hbm, xl_hbm, xr_hbm, ea_hbm, att_hbm, zn_hbm,
            ex_hbm, dp_hbm,
            srcv, dstv, xlr, xrr, ear, exb, accb, attv, sden,
            semI, semR, semL, semE):
        c = lax.axis_index("c")
        s = lax.axis_index("s")
        base_e, nch = _core_part(c, s, KA, EPWA0, EPWA1)
        pltpu.sync_copy(att_hbm, attv)
        pltpu.sync_copy(zn_hbm.at[pl.ds(s * NSL, NSL)],
                        sden.at[pl.ds(s * NSL, NSL)])
        plsc.subcore_barrier()

        def chunk(ci, j, b, pf1, pf2):
            o = 1 - b
            jn = (j + 1) % 4
            jp = (j + 2) % 4
            base = base_e + ci * KA
            if pf1:
                nb = base + KA
                pltpu.make_async_copy(
                    src_hbm.at[pl.ds(nb, KA)], srcv[jn], semI[jn]).wait()
                pltpu.make_async_copy(
                    dst_hbm.at[pl.ds(nb, KA)], dstv[jn], semI[jn]).wait()
                pltpu.async_copy(xl_hbm.at[srcv[jn]], xlr[o], semR[o])
                pltpu.async_copy(xr_hbm.at[dstv[jn]], xrr[o], semR[o])
                pltpu.async_copy(ea_hbm.at[pl.ds(nb, KA)], ear[o], semL[o])
            if pf2:
                nb2 = base + 2 * KA
                pltpu.async_copy(src_hbm.at[pl.ds(nb2, KA)], srcv[jp],
                                 semI[jp])
                pltpu.async_copy(dst_hbm.at[pl.ds(nb2, KA)], dstv[jp],
                                 semI[jp])
            pltpu.make_async_copy(xl_hbm.at[srcv[j]], xlr[b], semR[b]).wait()
            pltpu.make_async_copy(xr_hbm.at[dstv[j]], xrr[b], semR[b]).wait()
            pltpu.make_async_copy(
                ea_hbm.at[pl.ds(base, KA)], ear[b], semL[b]).wait()

            def edge(e, cc):
                acc = jnp.zeros((16,), _f32)
                for v in range(8):
                    sl = pl.ds(v * 16, 16)
                    m = xlr[b][e, sl] + xrr[b][e, sl] + ear[b][e, sl]
                    m = jnp.maximum(m, 0.0) + 0.2 * jnp.minimum(m, 0.0)
                    acc = acc + m * attv[sl]
                accb[pl.ds(e * 16, 16)] = acc
                return cc

            lax.fori_loop(0, KA, edge, 0)
            for g in range(KA // 16):
                rows = (g * 256 + 16 * lax.iota(_i32, 16))
                ssum = jnp.zeros((16,), _f32)
                for l in range(16):
                    ssum = ssum + plsc.load_gather(accb, [rows + l])
                exb[b][pl.ds(g * 16, 16)] = jnp.exp(ssum)
            dex = pltpu.async_copy(exb[b], ex_hbm.at[pl.ds(base, KA)], semE[b])
            pltpu.sync_copy(exb[b], sden.at[dstv[j]], add=True)
            dex.wait()

        pltpu.sync_copy(src_hbm.at[pl.ds(base_e, KA)], srcv[0])
        pltpu.sync_copy(dst_hbm.at[pl.ds(base_e, KA)], dstv[0])
        pltpu.async_copy(src_hbm.at[pl.ds(base_e + KA, KA)], srcv[1], semI[1])
        pltpu.async_copy(dst_hbm.at[pl.ds(base_e + KA, KA)], dstv[1], semI[1])
        pltpu.async_copy(xl_hbm.at[srcv[0]], xlr[0], semR[0])
        pltpu.async_copy(xr_hbm.at[dstv[0]], xrr[0], semR[0])
        pltpu.async_copy(ea_hbm.at[pl.ds(base_e, KA)], ear[0], semL[0])
        chunk(0, 0, 0, True, True)
        chunk(1, 1, 1, True, True)

        def mid(t, carry):
            ci = 2 + t * 4
            chunk(ci, 2, 0, True, True)
            chunk(ci + 1, 3, 1, True, True)
            chunk(ci + 2, 0, 0, True, True)
            chunk(ci + 3, 1, 1, True, True)
            return carry

        lax.fori_loop(0, (nch - 4) // 4, mid, 0)
        chunk(nch - 2, 2, 0, True, False)
        chunk(nch - 1, 3, 1, False, False)
        plsc.subcore_barrier()
        pltpu.sync_copy(sden.at[pl.ds(s * NSL, NSL)],
                        dp_hbm.at[c, pl.ds(s * NSL, NSL)])

    return sca(src, dst, xl, xr, ea, attf, zn)


def _sc_pass_b(src, dst, xl, ex, dp, zn, znc):
    @functools.partial(
        pl.kernel,
        mesh=_sc_mesh(),
        compiler_params=_SC_PARAMS,
        out_type=(
            jax.ShapeDtypeStruct((EP,), _f32),
            jax.ShapeDtypeStruct((NC, NPAD, C), _f32),
        ),
        scratch_types=[
            pltpu.VMEM((NPAD,), _f32),
            pltpu.VMEM((1024,), _f32),
            [pltpu.VMEM((KB,), _i32)] * 4,
            [pltpu.VMEM((KB,), _i32)] * 4,
            [pltpu.VMEM((KB,), _f32)] * 4,
            [pltpu.VMEM((KB,), _f32)] * 2,
            [pltpu.VMEM((KB, C), _f32)] * 2,
            [pltpu.VMEM((KB, C), _f32)] * 2,
            pltpu.VMEM_SHARED((NPAD, C), _f32),
            [pltpu.SemaphoreType.DMA] * 4,
            [pltpu.SemaphoreType.DMA] * 2,
            [pltpu.SemaphoreType.DMA] * 2,
            [pltpu.SemaphoreType.DMA] * 2,
        ],
    )
    def scb(src_hbm, dst_hbm, xl_hbm, ex_hbm, dp_hbm, zn_hbm, znc_hbm,
            alpha_hbm, hp_hbm,
            denv, tmpv, srcv, dstv, exv, alv, xlr, scl, sh,
            semI, semR, semA, semS):
        c = lax.axis_index("c")
        s = lax.axis_index("s")
        base_e, nch = _core_part(c, s, KB, EPWB0, EPWB1)
        pltpu.sync_copy(dp_hbm.at[0], denv)

        def addblk(bk, cc):
            pltpu.sync_copy(dp_hbm.at[1, pl.ds(bk * 1024, 1024)], tmpv)

            def addb(i, c2):
                dsl = pl.ds(bk * 1024 + i * 16, 16)
                denv[dsl] = denv[dsl] + tmpv[pl.ds(i * 16, 16)]
                return c2

            lax.fori_loop(0, 64, addb, 0)
            return cc

        lax.fori_loop(0, NPAD // 1024, addblk, 0)
        pltpu.sync_copy(znc_hbm.at[pl.ds(s * NSL, NSL)],
                        sh.at[pl.ds(s * NSL, NSL)])
        plsc.subcore_barrier()

        def chunk(ci, j, b, pf1, pf2):
            o = 1 - b
            jn = (j + 1) % 4
            jp = (j + 2) % 4
            base = base_e + ci * KB
            if pf1:
                nb = base + KB
                pltpu.make_async_copy(
                    src_hbm.at[pl.ds(nb, KB)], srcv[jn], semI[jn]).wait()
                pltpu.make_async_copy(
                    dst_hbm.at[pl.ds(nb, KB)], dstv[jn], semI[jn]).wait()
                pltpu.make_async_copy(
                    ex_hbm.at[pl.ds(nb, KB)], exv[jn], semI[jn]).wait()
                pltpu.async_copy(xl_hbm.at[srcv[jn]], xlr[o], semR[o])
            if pf2:
                nb2 = base + 2 * KB
                pltpu.async_copy(src_hbm.at[pl.ds(nb2, KB)], srcv[jp],
                                 semI[jp])
                pltpu.async_copy(dst_hbm.at[pl.ds(nb2, KB)], dstv[jp],
                                 semI[jp])
                pltpu.async_copy(ex_hbm.at[pl.ds(nb2, KB)], exv[jp],
                                 semI[jp])
            for g in range(KB // 16):
                sl = pl.ds(g * 16, 16)
                dsum = plsc.load_gather(denv, [dstv[j][sl]])
                alv[b][sl] = exv[j][sl] / (dsum + 1e-16)
            dal = pltpu.async_copy(
                alv[b], alpha_hbm.at[pl.ds(base, KB)], semA[b])
            pltpu.make_async_copy(xl_hbm.at[srcv[j]], xlr[b], semR[b]).wait()

            def edge(e, cc):
                ab = plsc.load_gather(alv[b], [lax.broadcast(e, (16,))])
                for v in range(8):
                    sl = pl.ds(v * 16, 16)
                    scl[b][e, sl] = xlr[b][e, sl] * ab
                return cc

            lax.fori_loop(0, KB, edge, 0)
            dsc = pltpu.async_copy(scl[b], sh.at[dstv[j]], semS[b], add=True)
            dal.wait()
            dsc.wait()

        pltpu.sync_copy(src_hbm.at[pl.ds(base_e, KB)], srcv[0])
        pltpu.sync_copy(dst_hbm.at[pl.ds(base_e, KB)], dstv[0])
        pltpu.sync_copy(ex_hbm.at[pl.ds(base_e, KB)], exv[0])
        pltpu.async_copy(src_hbm.at[pl.ds(base_e + KB, KB)], srcv[1], semI[1])
        pltpu.async_copy(dst_hbm.at[pl.ds(base_e + KB, KB)], dstv[1], semI[1])
        pltpu.async_copy(ex_hbm.at[pl.ds(base_e + KB, KB)], exv[1], semI[1])
        pltpu.async_copy(xl_hbm.at[srcv[0]], xlr[0], semR[0])
        chunk(0, 0, 0, True, True)
        chunk(1, 1, 1, True, True)

        def mid(t, carry):
            ci = 2 + t * 4
            chunk(ci, 2, 0, True, True)
            chunk(ci + 1, 3, 1, True, True)
            chunk(ci + 2, 0, 0, True, True)
            chunk(ci + 3, 1, 1, True, True)
            return carry

        lax.fori_loop(0, (nch - 4) // 4, mid, 0)
        chunk(nch - 2, 2, 0, True, False)
        chunk(nch - 1, 3, 1, False, False)
        plsc.subcore_barrier()
        pltpu.sync_copy(sh.at[pl.ds(s * NSL, NSL)],
                        hp_hbm.at[c, pl.ds(s * NSL, NSL)])

    return scb(src, dst, xl, ex, dp, zn, znc)


def _sc_pass_c(src, dst, hw, alpha, dis, znc):
    @functools.partial(
        pl.kernel,
        mesh=_sc_mesh(),
        compiler_params=_SC_PARAMS,
        out_type=jax.ShapeDtypeStruct((NC, NPAD, C), _f32),
        scratch_types=[
            pltpu.VMEM((NPAD,), _f32),
            [pltpu.VMEM((KB,), _i32)] * 4,
            [pltpu.VMEM((KB,), _i32)] * 4,
            [pltpu.VMEM((KB,), _f32)] * 4,
            [pltpu.VMEM((KB,), _f32)] * 2,
            [pltpu.VMEM((KB, C), _f32)] * 2,
            [pltpu.VMEM((KB, C), _f32)] * 2,
            pltpu.VMEM_SHARED((NPAD, C), _f32),
            [pltpu.SemaphoreType.DMA] * 4,
            [pltpu.SemaphoreType.DMA] * 2,
            [pltpu.SemaphoreType.DMA] * 2,
        ],
    )
    def scc(src_hbm, dst_hbm, hw_hbm, al_hbm, dis_hbm, znc_hbm, h2p_hbm,
            disv, srcv, dstv, alv, nrmv, hwr, scl, sh2,
            semI, semR, semS):
        c = lax.axis_index("c")
        s = lax.axis_index("s")
        base_e, nch = _core_part(c, s, KB, EPWB0, EPWB1)
        pltpu.sync_copy(dis_hbm, disv)
        pltpu.sync_copy(znc_hbm.at[pl.ds(s * NSL, NSL)],
                        sh2.at[pl.ds(s * NSL, NSL)])
        plsc.subcore_barrier()

        def chunk(ci, j, b, pf1, pf2):
            o = 1 - b
            jn = (j + 1) % 4
            jp = (j + 2) % 4
            base = base_e + ci * KB
            if pf1:
                nb = base + KB
                pltpu.make_async_copy(
                    src_hbm.at[pl.ds(nb, KB)], srcv[jn], semI[jn]).wait()
                pltpu.make_async_copy(
                    dst_hbm.at[pl.ds(nb, KB)], dstv[jn], semI[jn]).wait()
                pltpu.make_async_copy(
                    al_hbm.at[pl.ds(nb, KB)], alv[jn], semI[jn]).wait()
                pltpu.async_copy(hw_hbm.at[srcv[jn]], hwr[o], semR[o])
            if pf2:
                nb2 = base + 2 * KB
                pltpu.async_copy(src_hbm.at[pl.ds(nb2, KB)], srcv[jp],
                                 semI[jp])
                pltpu.async_copy(dst_hbm.at[pl.ds(nb2, KB)], dstv[jp],
                                 semI[jp])
                pltpu.async_copy(al_hbm.at[pl.ds(nb2, KB)], alv[jp],
                                 semI[jp])
            for g in range(KB // 16):
                sl = pl.ds(g * 16, 16)
                nrmv[b][sl] = (plsc.load_gather(disv, [srcv[j][sl]])
                               * alv[j][sl])
            pltpu.make_async_copy(hw_hbm.at[srcv[j]], hwr[b], semR[b]).wait()

            def edge(e, cc):
                nb16 = plsc.load_gather(nrmv[b], [lax.broadcast(e, (16,))])
                for v in range(8):
                    sl = pl.ds(v * 16, 16)
                    scl[b][e, sl] = hwr[b][e, sl] * nb16
                return cc

            lax.fori_loop(0, KB, edge, 0)
            dsc = pltpu.async_copy(scl[b], sh2.at[dstv[j]], semS[b], add=True)
            dsc.wait()

        pltpu.sync_copy(src_hbm.at[pl.ds(base_e, KB)], srcv[0])
        pltpu.sync_copy(dst_hbm.at[pl.ds(base_e, KB)], dstv[0])
        pltpu.sync_copy(al_hbm.at[pl.ds(base_e, KB)], alv[0])
        pltpu.async_copy(src_hbm.at[pl.ds(base_e + KB, KB)], srcv[1], semI[1])
        pltpu.async_copy(dst_hbm.at[pl.ds(base_e + KB, KB)], dstv[1], semI[1])
        pltpu.async_copy(al_hbm.at[pl.ds(base_e + KB, KB)], alv[1], semI[1])
        pltpu.async_copy(hw_hbm.at[srcv[0]], hwr[0], semR[0])
        chunk(0, 0, 0, True, True)
        chunk(1, 1, 1, True, True)

        def mid(t, carry):
            ci = 2 + t * 4
            chunk(ci, 2, 0, True, True)
            chunk(ci + 1, 3, 1, True, True)
            chunk(ci + 2, 0, 0, True, True)
            chunk(ci + 3, 1, 1, True, True)
            return carry

        lax.fori_loop(0, (nch - 4) // 4, mid, 0)
        chunk(nch - 2, 2, 0, True, False)
        chunk(nch - 1, 3, 1, False, False)
        plsc.subcore_barrier()
        pltpu.sync_copy(sh2.at[pl.ds(s * NSL, NSL)],
                        h2p_hbm.at[c, pl.ds(s * NSL, NSL)])

    return scc(src, dst, hw, alpha, dis, znc)


def kernel(x, edge_index, edge_feature, W_l, W_r, W_e, att, b_gat,
           W_gcn, b_gcn, W_out, b_out):
    pad_e = EP - E
    src = jnp.concatenate([edge_index[0], jnp.zeros((pad_e,), _i32)])
    dst = jnp.concatenate([edge_index[1], jnp.full((pad_e,), N, _i32)])
    efp = jnp.concatenate(
        [edge_feature, jnp.zeros((pad_e, D_EDGE), _f32)], axis=0)
    attf = att.reshape(C)
    zn = jnp.zeros((NPAD,), _f32)
    znc = jnp.zeros((NPAD, C), _f32)

    xl, xr = _tc_xlxr(x, W_l, W_r)
    ea = _tc_ea(efp, W_e)
    ex, dp = _sc_pass_a(src, dst, xl, xr, ea, attf, zn)
    alpha_full, hp = _sc_pass_b(src, dst, xl, ex, dp, zn, znc)
    hw, dis = _tc_mid(hp, b_gat, W_gcn, dp)
    h2p = _sc_pass_c(src, dst, hw, alpha_full, dis, znc)
    out_full = _tc_out(h2p, b_gcn, W_out, b_out)
    return (out_full[:N], alpha_full[:E].reshape(E, 1))


# speedup vs baseline: 1.0659x; 1.0659x over previous
import functools

import jax
import jax.numpy as jnp
from jax import lax
from jax.experimental import pallas as pl
from jax.experimental.pallas import tpu as pltpu
from jax.experimental.pallas import tpu_sc as plsc

N = 10000
E = 320000
C = 128
D_EDGE = 4
D_OUT = 2

NC = 2
NS = 16
NW = NC * NS
KA = 128
KB = 64
EPWA0 = 12288
EPWA1 = 8192
EPWB0 = 12288
EPWB1 = 8192
EP = NS * (EPWA0 + EPWA1)
NPAD = 10240
NSL = NPAD // NS

_f32 = jnp.float32
_i32 = jnp.int32


def _tc_xlxr(x, W_l, W_r):
    def body(x_ref, wl_ref, wr_ref, xl_ref, xr_ref):
        xb = x_ref[...]
        xl_ref[...] = jnp.dot(xb, wl_ref[...], preferred_element_type=_f32)
        xr_ref[...] = jnp.dot(xb, wr_ref[...], preferred_element_type=_f32)

    return pl.pallas_call(
        body,
        grid=(10,),
        in_specs=[
            pl.BlockSpec((1000, C), lambda i: (i, 0)),
            pl.BlockSpec((C, C), lambda i: (0, 0)),
            pl.BlockSpec((C, C), lambda i: (0, 0)),
        ],
        out_specs=[
            pl.BlockSpec((1000, C), lambda i: (i, 0)),
            pl.BlockSpec((1000, C), lambda i: (i, 0)),
        ],
        out_shape=[
            jax.ShapeDtypeStruct((N, C), _f32),
            jax.ShapeDtypeStruct((N, C), _f32),
        ],
    )(x, W_l, W_r)


def _tc_ea(efp, W_e):
    def body(ef_ref, we_ref, ea_ref):
        ea_ref[...] = jnp.dot(ef_ref[...], we_ref[...],
                              preferred_element_type=_f32)

    return pl.pallas_call(
        body,
        grid=(80,),
        in_specs=[
            pl.BlockSpec((EP // 80, D_EDGE), lambda i: (i, 0)),
            pl.BlockSpec((D_EDGE, C), lambda i: (0, 0)),
        ],
        out_specs=pl.BlockSpec((EP // 80, C), lambda i: (i, 0)),
        out_shape=jax.ShapeDtypeStruct((EP, C), _f32),
    )(efp, W_e)


def _tc_mid(hp, b_gat, W_gcn, denp):
    def body(hp_ref, bg_ref, wg_ref, dp_ref, hw_ref, dis_ref):
        h = jnp.maximum(hp_ref[0] + hp_ref[1] + bg_ref[...][None, :], 0.0)
        hw_ref[...] = jnp.dot(h, wg_ref[...], preferred_element_type=_f32)
        den = dp_ref[0] + dp_ref[1]
        dis_ref[...] = jnp.where(den > 0, 1.0, 0.0)

    return pl.pallas_call(
        body,
        grid=(10,),
        in_specs=[
            pl.BlockSpec((2, 1024, C), lambda i: (0, i, 0)),
            pl.BlockSpec((C,), lambda i: (0,)),
            pl.BlockSpec((C, C), lambda i: (0, 0)),
            pl.BlockSpec((2, 1024), lambda i: (0, i)),
        ],
        out_specs=[
            pl.BlockSpec((1024, C), lambda i: (i, 0)),
            pl.BlockSpec((1024,), lambda i: (i,)),
        ],
        out_shape=[
            jax.ShapeDtypeStruct((NPAD, C), _f32),
            jax.ShapeDtypeStruct((NPAD,), _f32),
        ],
    )(hp, b_gat, W_gcn, denp)


def _tc_out(h2p, b_gcn, W_out, b_out):
    def body(h2p_ref, bg_ref, wo_ref, bo_ref, out_ref):
        h2 = jnp.maximum(h2p_ref[0] + h2p_ref[1] + bg_ref[...][None, :], 0.0)
        out_ref[...] = (jnp.dot(h2, wo_ref[...], preferred_element_type=_f32)
                        + bo_ref[...][None, :])

    return pl.pallas_call(
        body,
        grid=(10,),
        in_specs=[
            pl.BlockSpec((2, 1024, C), lambda i: (0, i, 0)),
            pl.BlockSpec((C,), lambda i: (0,)),
            pl.BlockSpec((C, D_OUT), lambda i: (0, 0)),
            pl.BlockSpec((D_OUT,), lambda i: (0,)),
        ],
        out_specs=pl.BlockSpec((1024, D_OUT), lambda i: (i, 0)),
        out_shape=jax.ShapeDtypeStruct((NPAD, D_OUT), _f32),
    )(h2p, b_gcn, W_out, b_out)


def _sc_mesh():
    return plsc.VectorSubcoreMesh(core_axis_name="c", subcore_axis_name="s",
                                  num_cores=NC, num_subcores=NS)


_SC_PARAMS = pltpu.CompilerParams(needs_layout_passes=False)


def _core_part(c, s, k, epw0, epw1):
    base = jnp.where(c == 0, s * epw0, NS * epw0 + s * epw1)
    nch = jnp.where(c == 0, epw0 // k, epw1 // k)
    return base, nch


def _sc_pass_a(src, dst, xl, xr, ea, attf, zn):
    @functools.partial(
        pl.kernel,
        mesh=_sc_mesh(),
        compiler_params=_SC_PARAMS,
        out_type=(
            jax.ShapeDtypeStruct((EP,), _f32),
            jax.ShapeDtypeStruct((NC, NPAD), _f32),
        ),
        scratch_types=[
            [pltpu.VMEM((KA,), _i32)] * 4,
            [pltpu.VMEM((KA,), _i32)] * 4,
            [pltpu.VMEM((KA, C), _f32)] * 2,
            [pltpu.VMEM((KA, C), _f32)] * 2,
            [pltpu.VMEM((KA, C), _f32)] * 2,
            [pltpu.VMEM((KA,), _f32)] * 2,
            pltpu.VMEM((KA * 16,), _f32),
            pltpu.VMEM((C,), _f32),
            pltpu.VMEM_SHARED((NPAD,), _f32),
            [pltpu.SemaphoreType.DMA] * 4,
            [pltpu.SemaphoreType.DMA] * 2,
            [pltpu.SemaphoreType.DMA] * 2,
            [pltpu.SemaphoreType.DMA] * 2,
        ],
    )
    def sca(src_hbm, dst_hbm, xl_hbm, xr_hbm, ea_hbm, att_hbm, zn_hbm,
            ex_hbm, dp_hbm,
            srcv, dstv, xlr, xrr, ear, exb, accb, attv, sden,
            semI, semR, semL, semE):
        c = lax.axis_index("c")
        s = lax.axis_index("s")
        base_e, nch = _core_part(c, s, KA, EPWA0, EPWA1)
        pltpu.sync_copy(att_hbm, attv)
        pltpu.sync_copy(zn_hbm.at[pl.ds(s * NSL, NSL)],
                        sden.at[pl.ds(s * NSL, NSL)])
        plsc.subcore_barrier()

        def chunk(ci, j, b, pf1, pf2):
            o = 1 - b
            jn = (j + 1) % 4
            jp = (j + 2) % 4
            base = base_e + ci * KA
            if pf1:
                nb = base + KA
                pltpu.make_async_copy(
                    src_hbm.at[pl.ds(nb, KA)], srcv[jn], semI[jn]).wait()
                pltpu.make_async_copy(
                    dst_hbm.at[pl.ds(nb, KA)], dstv[jn], semI[jn]).wait()
                pltpu.async_copy(xl_hbm.at[srcv[jn]], xlr[o], semR[o])
                pltpu.async_copy(xr_hbm.at[dstv[jn]], xrr[o], semR[o])
                pltpu.async_copy(ea_hbm.at[pl.ds(nb, KA)], ear[o], semL[o])
            if pf2:
                nb2 = base + 2 * KA
                pltpu.async_copy(src_hbm.at[pl.ds(nb2, KA)], srcv[jp],
                                 semI[jp])
                pltpu.async_copy(dst_hbm.at[pl.ds(nb2, KA)], dstv[jp],
                                 semI[jp])
            pltpu.make_async_copy(xl_hbm.at[srcv[j]], xlr[b], semR[b]).wait()
            pltpu.make_async_copy(xr_hbm.at[dstv[j]], xrr[b], semR[b]).wait()
            pltpu.make_async_copy(
                ea_hbm.at[pl.ds(base, KA)], ear[b], semL[b]).wait()

            def edge(e, cc):
                acc = jnp.zeros((16,), _f32)
                for v in range(8):
                    sl = pl.ds(v * 16, 16)
                    m = xlr[b][e, sl] + xrr[b][e, sl] + ear[b][e, sl]
                    m = jnp.maximum(m, 0.0) + 0.2 * jnp.minimum(m, 0.0)
                    acc = acc + m * attv[sl]
                accb[pl.ds(e * 16, 16)] = acc
                return cc

            lax.fori_loop(0, KA, edge, 0)
            for g in range(KA // 16):
                rows = (g * 256 + 16 * lax.iota(_i32, 16))
                ssum = jnp.zeros((16,), _f32)
                for l in range(16):
                    ssum = ssum + plsc.load_gather(accb, [rows + l])
                exb[b][pl.ds(g * 16, 16)] = jnp.exp(ssum)
            dex = pltpu.async_copy(exb[b], ex_hbm.at[pl.ds(base, KA)], semE[b])
            pltpu.sync_copy(exb[b], sden.at[dstv[j]], add=True)
            dex.wait()

        pltpu.sync_copy(src_hbm.at[pl.ds(base_e, KA)], srcv[0])
        pltpu.sync_copy(dst_hbm.at[pl.ds(base_e, KA)], dstv[0])
        pltpu.async_copy(src_hbm.at[pl.ds(base_e + KA, KA)], srcv[1], semI[1])
        pltpu.async_copy(dst_hbm.at[pl.ds(base_e + KA, KA)], dstv[1], semI[1])
        pltpu.async_copy(xl_hbm.at[srcv[0]], xlr[0], semR[0])
        pltpu.async_copy(xr_hbm.at[dstv[0]], xrr[0], semR[0])
        pltpu.async_copy(ea_hbm.at[pl.ds(base_e, KA)], ear[0], semL[0])
        chunk(0, 0, 0, True, True)
        chunk(1, 1, 1, True, True)

        def mid(t, carry):
            ci = 2 + t * 4
            chunk(ci, 2, 0, True, True)
            chunk(ci + 1, 3, 1, True, True)
            chunk(ci + 2, 0, 0, True, True)
            chunk(ci + 3, 1, 1, True, True)
            return carry

        lax.fori_loop(0, (nch - 4) // 4, mid, 0)
        chunk(nch - 2, 2, 0, True, False)
        chunk(nch - 1, 3, 1, False, False)
        plsc.subcore_barrier()
        pltpu.sync_copy(sden.at[pl.ds(s * NSL, NSL)],
                        dp_hbm.at[c, pl.ds(s * NSL, NSL)])

    return sca(src, dst, xl, xr, ea, attf, zn)


def _sc_pass_b(src, dst, xl, ex, dp, zn, znc):
    @functools.partial(
        pl.kernel,
        mesh=_sc_mesh(),
        compiler_params=_SC_PARAMS,
        out_type=(
            jax.ShapeDtypeStruct((EP,), _f32),
            jax.ShapeDtypeStruct((NC, NPAD, C), _f32),
        ),
        scratch_types=[
            pltpu.VMEM((NPAD,), _f32),
            pltpu.VMEM((1024,), _f32),
            [pltpu.VMEM((KB,), _i32)] * 4,
            [pltpu.VMEM((KB,), _i32)] * 4,
            [pltpu.VMEM((KB,), _f32)] * 4,
            [pltpu.VMEM((KB,), _f32)] * 2,
            [pltpu.VMEM((KB, C), _f32)] * 2,
            [pltpu.VMEM((KB, C), _f32)] * 2,
            pltpu.VMEM_SHARED((NPAD, C), _f32),
            [pltpu.SemaphoreType.DMA] * 4,
            [pltpu.SemaphoreType.DMA] * 2,
            [pltpu.SemaphoreType.DMA] * 2,
            [pltpu.SemaphoreType.DMA] * 2,
        ],
    )
    def scb(src_hbm, dst_hbm, xl_hbm, ex_hbm, dp_hbm, zn_hbm, znc_hbm,
            alpha_hbm, hp_hbm,
            denv, tmpv, srcv, dstv, exv, alv, xlr, scl, sh,
            semI, semR, semA, semS):
        c = lax.axis_index("c")
        s = lax.axis_index("s")
        base_e, nch = _core_part(c, s, KB, EPWB0, EPWB1)
        pltpu.sync_copy(dp_hbm.at[0], denv)

        def addblk(bk, cc):
            pltpu.sync_copy(dp_hbm.at[1, pl.ds(bk * 1024, 1024)], tmpv)

            def addb(i, c2):
                dsl = pl.ds(bk * 1024 + i * 16, 16)
                denv[dsl] = denv[dsl] + tmpv[pl.ds(i * 16, 16)]
                return c2

            lax.fori_loop(0, 64, addb, 0)
            return cc

        lax.fori_loop(0, NPAD // 1024, addblk, 0)
        pltpu.sync_copy(znc_hbm.at[pl.ds(s * NSL, NSL)],
                        sh.at[pl.ds(s * NSL, NSL)])
        plsc.subcore_barrier()

        def chunk(ci, j, b, pf1, pf2):
            o = 1 - b
            jn = (j + 1) % 4
            jp = (j + 2) % 4
            base = base_e + ci * KB
            if pf1:
                nb = base + KB
                pltpu.make_async_copy(
                    src_hbm.at[pl.ds(nb, KB)], srcv[jn], semI[jn]).wait()
                pltpu.make_async_copy(
                    dst_hbm.at[pl.ds(nb, KB)], dstv[jn], semI[jn]).wait()
                pltpu.make_async_copy(
                    ex_hbm.at[pl.ds(nb, KB)], exv[jn], semI[jn]).wait()
                pltpu.async_copy(xl_hbm.at[srcv[jn]], xlr[o], semR[o])
            if pf2:
                nb2 = base + 2 * KB
                pltpu.async_copy(src_hbm.at[pl.ds(nb2, KB)], srcv[jp],
                                 semI[jp])
                pltpu.async_copy(dst_hbm.at[pl.ds(nb2, KB)], dstv[jp],
                                 semI[jp])
                pltpu.async_copy(ex_hbm.at[pl.ds(nb2, KB)], exv[jp],
                                 semI[jp])
            for g in range(KB // 16):
                sl = pl.ds(g * 16, 16)
                dsum = plsc.load_gather(denv, [dstv[j][sl]])
                alv[b][sl] = exv[j][sl] / (dsum + 1e-16)
            dal = pltpu.async_copy(
                alv[b], alpha_hbm.at[pl.ds(base, KB)], semA[b])
            pltpu.make_async_copy(xl_hbm.at[srcv[j]], xlr[b], semR[b]).wait()

            def edge(e, cc):
                ab = plsc.load_gather(alv[b], [lax.broadcast(e, (16,))])
                for v in range(8):
                    sl = pl.ds(v * 16, 16)
                    scl[b][e, sl] = xlr[b][e, sl] * ab
                return cc

            lax.fori_loop(0, KB, edge, 0)
            dsc = pltpu.async_copy(scl[b], sh.at[dstv[j]], semS[b], add=True)
            dal.wait()
            dsc.wait()

        pltpu.sync_copy(src_hbm.at[pl.ds(base_e, KB)], srcv[0])
        pltpu.sync_copy(dst_hbm.at[pl.ds(base_e, KB)], dstv[0])
        pltpu.sync_copy(ex_hbm.at[pl.ds(base_e, KB)], exv[0])
        pltpu.async_copy(src_hbm.at[pl.ds(base_e + KB, KB)], srcv[1], semI[1])
        pltpu.async_copy(dst_hbm.at[pl.ds(base_e + KB, KB)], dstv[1], semI[1])
        pltpu.async_copy(ex_hbm.at[pl.ds(base_e + KB, KB)], exv[1], semI[1])
        pltpu.async_copy(xl_hbm.at[srcv[0]], xlr[0], semR[0])
        chunk(0, 0, 0, True, True)
        chunk(1, 1, 1, True, True)

        def mid(t, carry):
            ci = 2 + t * 4
            chunk(ci, 2, 0, True, True)
            chunk(ci + 1, 3, 1, True, True)
            chunk(ci + 2, 0, 0, True, True)
            chunk(ci + 3, 1, 1, True, True)
            return carry

        lax.fori_loop(0, (nch - 4) // 4, mid, 0)
        chunk(nch - 2, 2, 0, True, False)
        chunk(nch - 1, 3, 1, False, False)
        plsc.subcore_barrier()
        pltpu.sync_copy(sh.at[pl.ds(s * NSL, NSL)],
                        hp_hbm.at[c, pl.ds(s * NSL, NSL)])

    return scb(src, dst, xl, ex, dp, zn, znc)


def _sc_pass_c(src, dst, hw, alpha, dis, znc):
    @functools.partial(
        pl.kernel,
        mesh=_sc_mesh(),
        compiler_params=_SC_PARAMS,
        out_type=jax.ShapeDtypeStruct((NC, NPAD, C), _f32),
        scratch_types=[
            pltpu.VMEM((NPAD,), _f32),
            [pltpu.VMEM((KB,), _i32)] * 4,
            [pltpu.VMEM((KB,), _i32)] * 4,
            [pltpu.VMEM((KB,), _f32)] * 4,
            [pltpu.VMEM((KB,), _f32)] * 2,
            [pltpu.VMEM((KB, C), _f32)] * 2,
            [pltpu.VMEM((KB, C), _f32)] * 2,
            pltpu.VMEM_SHARED((NPAD, C), _f32),
            [pltpu.SemaphoreType.DMA] * 4,
            [pltpu.SemaphoreType.DMA] * 2,
            [pltpu.SemaphoreType.DMA] * 2,
        ],
    )
    def scc(src_hbm, dst_hbm, hw_hbm, al_hbm, dis_hbm, znc_hbm, h2p_hbm,
            disv, srcv, dstv, alv, nrmv, hwr, scl, sh2,
            semI, semR, semS):
        c = lax.axis_index("c")
        s = lax.axis_index("s")
        base_e, nch = _core_part(c, s, KB, EPWB0, EPWB1)
        pltpu.sync_copy(dis_hbm, disv)
        pltpu.sync_copy(znc_hbm.at[pl.ds(s * NSL, NSL)],
                        sh2.at[pl.ds(s * NSL, NSL)])
        plsc.subcore_barrier()

        def chunk(ci, j, b, pf1, pf2):
            o = 1 - b
            jn = (j + 1) % 4
            jp = (j + 2) % 4
            base = base_e + ci * KB
            if pf1:
                nb = base + KB
                pltpu.make_async_copy(
                    src_hbm.at[pl.ds(nb, KB)], srcv[jn], semI[jn]).wait()
                pltpu.make_async_copy(
                    dst_hbm.at[pl.ds(nb, KB)], dstv[jn], semI[jn]).wait()
                pltpu.make_async_copy(
                    al_hbm.at[pl.ds(nb, KB)], alv[jn], semI[jn]).wait()
                pltpu.async_copy(hw_hbm.at[srcv[jn]], hwr[o], semR[o])
            if pf2:
                nb2 = base + 2 * KB
                pltpu.async_copy(src_hbm.at[pl.ds(nb2, KB)], srcv[jp],
                                 semI[jp])
                pltpu.async_copy(dst_hbm.at[pl.ds(nb2, KB)], dstv[jp],
                                 semI[jp])
                pltpu.async_copy(al_hbm.at[pl.ds(nb2, KB)], alv[jp],
                                 semI[jp])
            for g in range(KB // 16):
                sl = pl.ds(g * 16, 16)
                nrmv[b][sl] = (plsc.load_gather(disv, [srcv[j][sl]])
                               * alv[j][sl])
            pltpu.make_async_copy(hw_hbm.at[srcv[j]], hwr[b], semR[b]).wait()

            def edge(e, cc):
                nb16 = plsc.load_gather(nrmv[b], [lax.broadcast(e, (16,))])
                for v in range(8):
                    sl = pl.ds(v * 16, 16)
                    scl[b][e, sl] = hwr[b][e, sl] * nb16
                return cc

            lax.fori_loop(0, KB, edge, 0)
            dsc = pltpu.async_copy(scl[b], sh2.at[dstv[j]], semS[b], add=True)
            dsc.wait()

        pltpu.sync_copy(src_hbm.at[pl.ds(base_e, KB)], srcv[0])
        pltpu.sync_copy(dst_hbm.at[pl.ds(base_e, KB)], dstv[0])
        pltpu.sync_copy(al_hbm.at[pl.ds(base_e, KB)], alv[0])
        pltpu.async_copy(src_hbm.at[pl.ds(base_e + KB, KB)], srcv[1], semI[1])
        pltpu.async_copy(dst_hbm.at[pl.ds(base_e + KB, KB)], dstv[1], semI[1])
        pltpu.async_copy(al_hbm.at[pl.ds(base_e + KB, KB)], alv[1], semI[1])
        pltpu.async_copy(hw_hbm.at[srcv[0]], hwr[0], semR[0])
        chunk(0, 0, 0, True, True)
        chunk(1, 1, 1, True, True)

        def mid(t, carry):
            ci = 2 + t * 4
            chunk(ci, 2, 0, True, True)
            chunk(ci + 1, 3, 1, True, True)
            chunk(ci + 2, 0, 0, True, True)
            chunk(ci + 3, 1, 1, True, True)
            return carry

        lax.fori_loop(0, (nch - 4) // 4, mid, 0)
        chunk(nch - 2, 2, 0, True, False)
        chunk(nch - 1, 3, 1, False, False)
        plsc.subcore_barrier()
        pltpu.sync_copy(sh2.at[pl.ds(s * NSL, NSL)],
                        h2p_hbm.at[c, pl.ds(s * NSL, NSL)])

    return scc(src, dst, hw, alpha, dis, znc)


def kernel(x, edge_index, edge_feature, W_l, W_r, W_e, att, b_gat,
           W_gcn, b_gcn, W_out, b_out):
    pad_e = EP - E
    src = jnp.concatenate([edge_index[0], jnp.zeros((pad_e,), _i32)])
    dst = jnp.concatenate([edge_index[1], jnp.full((pad_e,), N, _i32)])
    efp = jnp.concatenate(
        [edge_feature, jnp.zeros((pad_e, D_EDGE), _f32)], axis=0)
    attf = att.reshape(C)
    zn = jnp.zeros((NPAD,), _f32)
    znc = jnp.zeros((NPAD, C), _f32)

    xl, xr = _tc_xlxr(x, W_l, W_r)
    ea = _tc_ea(efp, W_e)
    ex, dp = _sc_pass_a(src, dst, xl, xr, ea, attf, zn)
    alpha_full, hp = _sc_pass_b(src, dst, xl, ex, dp, zn, znc)
    hw, dis = _tc_mid(hp, b_gat, W_gcn, dp)
    h2p = _sc_pass_c(src, dst, hw, alpha_full, dis, znc)
    out_full = _tc_out(h2p, b_gcn, W_out, b_out)
    return (out_full[:N], alpha_full[:E].reshape(E, 1))
